# baseline (speedup 1.0000x reference)
import jax
import jax.numpy as jnp
from jax import lax
from jax.experimental import pallas as pl
from jax.experimental.pallas import tpu as pltpu
from jax.experimental.pallas import tpu_sc as plsc

_NC = 2
_NS = 16
_NW = _NC * _NS

_U_SHARD, _U_CHUNK = 31248, 504
_I_SHARD, _I_CHUNK = 3120, 312


def _stream_copy(src, dst, base, n, chunk, bufs, isems, osems):
    def in_cp(k, b):
        return pltpu.make_async_copy(
            src.at[pl.ds(base + k * chunk, chunk), :],
            bufs[b].at[pl.ds(0, chunk), :], isems[b])

    def out_cp(k, b):
        return pltpu.make_async_copy(
            bufs[b].at[pl.ds(0, chunk), :],
            dst.at[pl.ds(base + k * chunk, chunk), :], osems[b])

    in_cp(0, 0).start()

    @pl.loop(0, n, step=2)
    def _pair(g):
        for b in range(2):
            k = g + b
            in_cp(k, b).wait()
            out_cp(k, b).start()
            nxt = k + 1

            @pl.when(nxt < n)
            def _start_next():
                @pl.when(k >= 1)
                def _drain_prev():
                    out_cp(k - 1, 1 - b).wait()

                in_cp(nxt, 1 - b).start()

    out_cp(n - 2, 0).wait()
    out_cp(n - 1, 1).wait()


def _tail_copy(src, dst, base, rows, buf, isem, osem):
    pltpu.make_async_copy(
        src.at[pl.ds(base, rows), :], buf.at[pl.ds(0, rows), :], isem).start()
    pltpu.make_async_copy(
        src.at[pl.ds(base, rows), :], buf.at[pl.ds(0, rows), :], isem).wait()
    pltpu.make_async_copy(
        buf.at[pl.ds(0, rows), :], dst.at[pl.ds(base, rows), :], osem).start()
    pltpu.make_async_copy(
        buf.at[pl.ds(0, rows), :], dst.at[pl.ds(base, rows), :], osem).wait()


def _copy_body(u_hbm, i_hbm, uo_hbm, io_hbm, buf0, buf1, is0, is1, os0, os1):
    wid = lax.axis_index("s") * _NC + lax.axis_index("c")
    bufs, isems, osems = (buf0, buf1), (is0, is1), (os0, os1)

    _stream_copy(u_hbm, uo_hbm, wid * _U_SHARD, _U_SHARD // _U_CHUNK,
                 _U_CHUNK, bufs, isems, osems)
    _stream_copy(i_hbm, io_hbm, wid * _I_SHARD, _I_SHARD // _I_CHUNK,
                 _I_CHUNK, bufs, isems, osems)

    n_users, n_items = u_hbm.shape[0], i_hbm.shape[0]

    @pl.when(wid == 0)
    def _tails():
        _tail_copy(u_hbm, uo_hbm, _NW * _U_SHARD, n_users - _NW * _U_SHARD,
                   buf0, is0, os0)
        _tail_copy(i_hbm, io_hbm, _NW * _I_SHARD, n_items - _NW * _I_SHARD,
                   buf0, is0, os0)


def kernel(user_embeds, item_embeds, adj):
    d = user_embeds.shape[1]
    sc_copy = pl.kernel(
        _copy_body,
        out_type=(
            jax.ShapeDtypeStruct(user_embeds.shape, user_embeds.dtype),
            jax.ShapeDtypeStruct(item_embeds.shape, item_embeds.dtype),
        ),
        mesh=plsc.VectorSubcoreMesh(core_axis_name="c", subcore_axis_name="s"),
        scratch_types=[
            pltpu.VMEM((_U_CHUNK, d), jnp.float32),
            pltpu.VMEM((_U_CHUNK, d), jnp.float32),
            pltpu.SemaphoreType.DMA,
            pltpu.SemaphoreType.DMA,
            pltpu.SemaphoreType.DMA,
            pltpu.SemaphoreType.DMA,
        ],
    )
    return sc_copy(user_embeds, item_embeds)
